# baseline (device time: 208596 ns/iter reference)
import functools

import jax
import jax.numpy as jnp
from jax import lax
from jax.experimental import pallas as pl
from jax.experimental.pallas import tpu as pltpu

N_DEV = 8
M_PER = 512
HALF = M_PER // 2
N_CHAIN = 4
QTR = HALF // N_CHAIN
N_HOP = N_DEV - 1


def kernel(x, w_mat):
    m_per, k = x.shape
    _, n_per = w_mat.shape
    assert m_per == M_PER and k == 4096 and n_per == 1024

    x = x.astype(jnp.bfloat16)
    w_mat = w_mat.astype(jnp.bfloat16)

    def body(x_ref, w_ref, out_ref, cw_buf, ccw_buf,
             cw_send, cw_recv, ccw_send, ccw_recv):
        me = lax.axis_index("i")
        right = lax.rem(me + 1, N_DEV)
        left = lax.rem(me + N_DEV - 1, N_DEV)

        barrier_sem = pltpu.get_barrier_semaphore()
        for nbr in (left, right):
            pl.semaphore_signal(
                barrier_sem, inc=1,
                device_id=(nbr,), device_id_type=pl.DeviceIdType.MESH,
            )
        pl.semaphore_wait(barrier_sem, 2)

        def send(src_ref, dst_ref, s_sem, r_sem, dev):
            rdma = pltpu.make_async_remote_copy(
                src_ref=src_ref, dst_ref=dst_ref,
                send_sem=s_sem, recv_sem=r_sem,
                device_id=(dev,), device_id_type=pl.DeviceIdType.MESH,
            )
            rdma.start()
            return rdma

        sends = []
        for c in range(N_CHAIN):
            sends.append(send(x_ref.at[pl.ds(c * QTR, QTR), :],
                              cw_buf.at[0, c], cw_send.at[0, c],
                              cw_recv.at[0, c], right))
            sends.append(send(x_ref.at[pl.ds(HALF + c * QTR, QTR), :],
                              ccw_buf.at[0, c], ccw_send.at[0, c],
                              ccw_recv.at[0, c], left))

        out_ref[pl.ds(me * M_PER, M_PER), :] = jnp.dot(
            x_ref[:, :], w_ref[:, :], preferred_element_type=jnp.float32
        )

        def recv_wait(buf, s_sems, r_sems, h, c, dev):
            pltpu.make_async_remote_copy(
                src_ref=buf.at[h, c], dst_ref=buf.at[h, c],
                send_sem=s_sems.at[h, c], recv_sem=r_sems.at[h, c],
                device_id=(dev,), device_id_type=pl.DeviceIdType.MESH,
            ).wait_recv()

        for h in range(N_HOP):
            for c in range(N_CHAIN):
                recv_wait(cw_buf, cw_send, cw_recv, h, c, right)
                if h + 1 < N_HOP:
                    sends.append(send(cw_buf.at[h, c], cw_buf.at[h + 1, c],
                                      cw_send.at[h + 1, c],
                                      cw_recv.at[h + 1, c], right))
            for c in range(N_CHAIN):
                recv_wait(ccw_buf, ccw_send, ccw_recv, h, c, left)
                if h + 1 < N_HOP:
                    sends.append(send(ccw_buf.at[h, c], ccw_buf.at[h + 1, c],
                                      ccw_send.at[h + 1, c],
                                      ccw_recv.at[h + 1, c], left))

            o_cw = lax.rem(me + 2 * N_DEV - 1 - h, N_DEV)
            out_ref[pl.ds(o_cw * M_PER, HALF), :] = jnp.dot(
                cw_buf[h].reshape(HALF, k), w_ref[:, :],
                preferred_element_type=jnp.float32,
            )
            o_ccw = lax.rem(me + 1 + h, N_DEV)
            out_ref[pl.ds(o_ccw * M_PER + HALF, HALF), :] = jnp.dot(
                ccw_buf[h].reshape(HALF, k), w_ref[:, :],
                preferred_element_type=jnp.float32,
            )

        for rdma in sends:
            rdma.wait_send()

    out_shape = jax.ShapeDtypeStruct((N_DEV * M_PER, n_per), jnp.float32)
    return pl.pallas_call(
        body,
        out_shape=out_shape,
        in_specs=[
            pl.BlockSpec(memory_space=pltpu.VMEM),
            pl.BlockSpec(memory_space=pltpu.VMEM),
        ],
        out_specs=pl.BlockSpec(memory_space=pltpu.VMEM),
        scratch_shapes=[
            pltpu.VMEM((N_HOP, N_CHAIN, QTR, k), jnp.bfloat16),
            pltpu.VMEM((N_HOP, N_CHAIN, QTR, k), jnp.bfloat16),
            pltpu.SemaphoreType.DMA((N_HOP, N_CHAIN)),
            pltpu.SemaphoreType.DMA((N_HOP, N_CHAIN)),
            pltpu.SemaphoreType.DMA((N_HOP, N_CHAIN)),
            pltpu.SemaphoreType.DMA((N_HOP, N_CHAIN)),
        ],
        compiler_params=pltpu.CompilerParams(
            collective_id=0,
            vmem_limit_bytes=64 * 1024 * 1024,
        ),
    )(x, w_mat)


# device time: 203425 ns/iter; 1.0254x vs baseline; 1.0254x over previous
import functools

import jax
import jax.numpy as jnp
from jax import lax
from jax.experimental import pallas as pl
from jax.experimental.pallas import tpu as pltpu

N_DEV = 8
M_PER = 512
HALF = M_PER // 2
N_CHAIN = 2
QTR = HALF // N_CHAIN
N_HOP = N_DEV - 1


def kernel(x, w_mat):
    m_per, k = x.shape
    _, n_per = w_mat.shape
    assert m_per == M_PER and k == 4096 and n_per == 1024

    x = x.astype(jnp.bfloat16)
    w_mat = w_mat.astype(jnp.bfloat16)

    def body(x_ref, w_ref, out_ref, cw_buf, ccw_buf,
             cw_send, cw_recv, ccw_send, ccw_recv):
        me = lax.axis_index("i")
        right = lax.rem(me + 1, N_DEV)
        left = lax.rem(me + N_DEV - 1, N_DEV)

        barrier_sem = pltpu.get_barrier_semaphore()
        for nbr in (left, right):
            pl.semaphore_signal(
                barrier_sem, inc=1,
                device_id=(nbr,), device_id_type=pl.DeviceIdType.MESH,
            )
        pl.semaphore_wait(barrier_sem, 2)

        def send(src_ref, dst_ref, s_sem, r_sem, dev):
            rdma = pltpu.make_async_remote_copy(
                src_ref=src_ref, dst_ref=dst_ref,
                send_sem=s_sem, recv_sem=r_sem,
                device_id=(dev,), device_id_type=pl.DeviceIdType.MESH,
            )
            rdma.start()
            return rdma

        sends = []
        for c in range(N_CHAIN):
            sends.append(send(x_ref.at[pl.ds(c * QTR, QTR), :],
                              cw_buf.at[0, c], cw_send.at[0, c],
                              cw_recv.at[0, c], right))
            sends.append(send(x_ref.at[pl.ds(HALF + c * QTR, QTR), :],
                              ccw_buf.at[0, c], ccw_send.at[0, c],
                              ccw_recv.at[0, c], left))

        out_ref[pl.ds(me * M_PER, M_PER), :] = jnp.dot(
            x_ref[:, :], w_ref[:, :], preferred_element_type=jnp.float32
        )

        def recv_wait(buf, s_sems, r_sems, h, c, dev):
            pltpu.make_async_remote_copy(
                src_ref=buf.at[h, c], dst_ref=buf.at[h, c],
                send_sem=s_sems.at[h, c], recv_sem=r_sems.at[h, c],
                device_id=(dev,), device_id_type=pl.DeviceIdType.MESH,
            ).wait_recv()

        for h in range(N_HOP):
            for c in range(N_CHAIN):
                recv_wait(cw_buf, cw_send, cw_recv, h, c, right)
                if h + 1 < N_HOP:
                    sends.append(send(cw_buf.at[h, c], cw_buf.at[h + 1, c],
                                      cw_send.at[h + 1, c],
                                      cw_recv.at[h + 1, c], right))
            for c in range(N_CHAIN):
                recv_wait(ccw_buf, ccw_send, ccw_recv, h, c, left)
                if h + 1 < N_HOP:
                    sends.append(send(ccw_buf.at[h, c], ccw_buf.at[h + 1, c],
                                      ccw_send.at[h + 1, c],
                                      ccw_recv.at[h + 1, c], left))


        for rdma in sends:
            rdma.wait_send()

    out_shape = jax.ShapeDtypeStruct((N_DEV * M_PER, n_per), jnp.float32)
    return pl.pallas_call(
        body,
        out_shape=out_shape,
        in_specs=[
            pl.BlockSpec(memory_space=pltpu.VMEM),
            pl.BlockSpec(memory_space=pltpu.VMEM),
        ],
        out_specs=pl.BlockSpec(memory_space=pltpu.VMEM),
        scratch_shapes=[
            pltpu.VMEM((N_HOP, N_CHAIN, QTR, k), jnp.bfloat16),
            pltpu.VMEM((N_HOP, N_CHAIN, QTR, k), jnp.bfloat16),
            pltpu.SemaphoreType.DMA((N_HOP, N_CHAIN)),
            pltpu.SemaphoreType.DMA((N_HOP, N_CHAIN)),
            pltpu.SemaphoreType.DMA((N_HOP, N_CHAIN)),
            pltpu.SemaphoreType.DMA((N_HOP, N_CHAIN)),
        ],
        compiler_params=pltpu.CompilerParams(
            collective_id=0,
            vmem_limit_bytes=64 * 1024 * 1024,
        ),
    )(x, w_mat)


# device time: 191816 ns/iter; 1.0875x vs baseline; 1.0605x over previous
import jax
import jax.numpy as jnp
from jax import lax
from jax.experimental import pallas as pl
from jax.experimental.pallas import tpu as pltpu

N_DEV = 8
M_PER = 512
HALF = M_PER // 2
N_CHAIN = 2
QTR = HALF // N_CHAIN
N_HOP = N_DEV - 1
N_STAGE = 4


def kernel(x, w_mat):
    m_per, k = x.shape
    _, n_per = w_mat.shape
    assert m_per == M_PER and k == 4096 and n_per == 1024

    x = x.astype(jnp.bfloat16)

    def body(x_ref, w_hbm, out_hbm, cw_buf, ccw_buf, w_f32, w_bf, stage,
             cw_send, cw_recv, ccw_send, ccw_recv, w_sem, out_sems):
        me = lax.axis_index("i")
        right = lax.rem(me + 1, N_DEV)
        left = lax.rem(me + N_DEV - 1, N_DEV)

        w_dma = pltpu.make_async_copy(w_hbm, w_f32, w_sem)
        w_dma.start()

        barrier_sem = pltpu.get_barrier_semaphore()
        for nbr in (left, right):
            pl.semaphore_signal(
                barrier_sem, inc=1,
                device_id=(nbr,), device_id_type=pl.DeviceIdType.MESH,
            )
        pl.semaphore_wait(barrier_sem, 2)

        def send(src_ref, dst_ref, s_sem, r_sem, dev):
            rdma = pltpu.make_async_remote_copy(
                src_ref=src_ref, dst_ref=dst_ref,
                send_sem=s_sem, recv_sem=r_sem,
                device_id=(dev,), device_id_type=pl.DeviceIdType.MESH,
            )
            rdma.start()
            return rdma

        sends = []
        for c in range(N_CHAIN):
            sends.append(send(x_ref.at[pl.ds(c * QTR, QTR), :],
                              cw_buf.at[0, c], cw_send.at[0, c],
                              cw_recv.at[0, c], right))
            sends.append(send(x_ref.at[pl.ds(HALF + c * QTR, QTR), :],
                              ccw_buf.at[0, c], ccw_send.at[0, c],
                              ccw_recv.at[0, c], left))

        w_dma.wait()
        w_bf[:, :] = w_f32[:, :].astype(jnp.bfloat16)

        out_copies = {}

        def emit(block_idx, row_start, value):
            slot = block_idx % N_STAGE
            if slot in out_copies:
                out_copies[slot].wait()
            stage[slot, :, :] = value
            cp = pltpu.make_async_copy(
                stage.at[slot],
                out_hbm.at[pl.ds(row_start, HALF), :],
                out_sems.at[slot],
            )
            cp.start()
            out_copies[slot] = cp

        emit(0, me * M_PER, jnp.dot(
            x_ref[pl.ds(0, HALF), :], w_bf[:, :],
            preferred_element_type=jnp.float32))
        emit(1, me * M_PER + HALF, jnp.dot(
            x_ref[pl.ds(HALF, HALF), :], w_bf[:, :],
            preferred_element_type=jnp.float32))

        def recv_wait(buf, s_sems, r_sems, h, c, dev):
            pltpu.make_async_remote_copy(
                src_ref=buf.at[h, c], dst_ref=buf.at[h, c],
                send_sem=s_sems.at[h, c], recv_sem=r_sems.at[h, c],
                device_id=(dev,), device_id_type=pl.DeviceIdType.MESH,
            ).wait_recv()

        for h in range(N_HOP):
            for c in range(N_CHAIN):
                recv_wait(cw_buf, cw_send, cw_recv, h, c, right)
                if h + 1 < N_HOP:
                    sends.append(send(cw_buf.at[h, c], cw_buf.at[h + 1, c],
                                      cw_send.at[h + 1, c],
                                      cw_recv.at[h + 1, c], right))
            for c in range(N_CHAIN):
                recv_wait(ccw_buf, ccw_send, ccw_recv, h, c, left)
                if h + 1 < N_HOP:
                    sends.append(send(ccw_buf.at[h, c], ccw_buf.at[h + 1, c],
                                      ccw_send.at[h + 1, c],
                                      ccw_recv.at[h + 1, c], left))

            o_cw = lax.rem(me + 2 * N_DEV - 1 - h, N_DEV)
            emit(2 + 2 * h, o_cw * M_PER, jnp.dot(
                cw_buf[h].reshape(HALF, k), w_bf[:, :],
                preferred_element_type=jnp.float32))
            o_ccw = lax.rem(me + 1 + h, N_DEV)
            emit(3 + 2 * h, o_ccw * M_PER + HALF, jnp.dot(
                ccw_buf[h].reshape(HALF, k), w_bf[:, :],
                preferred_element_type=jnp.float32))

        for rdma in sends:
            rdma.wait_send()
        for cp in out_copies.values():
            cp.wait()

    out_shape = jax.ShapeDtypeStruct((N_DEV * M_PER, n_per), jnp.float32)
    return pl.pallas_call(
        body,
        out_shape=out_shape,
        in_specs=[
            pl.BlockSpec(memory_space=pltpu.VMEM),
            pl.BlockSpec(memory_space=pl.ANY),
        ],
        out_specs=pl.BlockSpec(memory_space=pl.ANY),
        scratch_shapes=[
            pltpu.VMEM((N_HOP, N_CHAIN, QTR, k), jnp.bfloat16),
            pltpu.VMEM((N_HOP, N_CHAIN, QTR, k), jnp.bfloat16),
            pltpu.VMEM((k, 1024), jnp.float32),
            pltpu.VMEM((k, 1024), jnp.bfloat16),
            pltpu.VMEM((N_STAGE, HALF, 1024), jnp.float32),
            pltpu.SemaphoreType.DMA((N_HOP, N_CHAIN)),
            pltpu.SemaphoreType.DMA((N_HOP, N_CHAIN)),
            pltpu.SemaphoreType.DMA((N_HOP, N_CHAIN)),
            pltpu.SemaphoreType.DMA((N_HOP, N_CHAIN)),
            pltpu.SemaphoreType.DMA,
            pltpu.SemaphoreType.DMA((N_STAGE,)),
        ],
        compiler_params=pltpu.CompilerParams(
            collective_id=0,
            vmem_limit_bytes=64 * 1024 * 1024,
        ),
    )(x, w_mat)
